# baseline (speedup 1.0000x reference)
import functools

import jax
import jax.numpy as jnp
from jax import lax
from jax.experimental import pallas as pl
from jax.experimental.pallas import tpu as pltpu
from jax.experimental.pallas import tpu_sc as plsc

EMB = 64
BATCH = 16384
NC = 2
NS = 16
NW = NC * NS
BPW = BATCH // NW
CHUNK = 128
NCHUNK = BPW // CHUNK
NGROUP = BPW // 16
LANES = 16
PSTRIDE = LANES + 1


def _amf_body(a_ref, b_ref, tab_ref, btab_ref, wv_ref, w0b_ref, out_ref,
              idx_a, idx_b, rows_a, rows_b, ba, bb, wv_v, w0b_v, ps, out_v,
              sem):
    wid = lax.axis_index("s") * NC + lax.axis_index("c")
    base = wid * BPW

    pltpu.sync_copy(a_ref.at[pl.ds(wid * NCHUNK, NCHUNK)], idx_a)
    pltpu.sync_copy(b_ref.at[pl.ds(wid * NCHUNK, NCHUNK)], idx_b)
    pltpu.sync_copy(wv_ref, wv_v)
    pltpu.sync_copy(w0b_ref, w0b_v)

    copies = []
    for k in range(NCHUNK):
        copies.append(pltpu.async_copy(
            tab_ref.at[idx_a.at[k]], rows_a.at[pl.ds(k * CHUNK, CHUNK)], sem))
        copies.append(pltpu.async_copy(
            tab_ref.at[idx_b.at[k]], rows_b.at[pl.ds(k * CHUNK, CHUNK)], sem))
        copies.append(pltpu.async_copy(
            btab_ref.at[idx_a.at[k]], ba.at[pl.ds(k * CHUNK, CHUNK)], sem))
        copies.append(pltpu.async_copy(
            btab_ref.at[idx_b.at[k]], bb.at[pl.ds(k * CHUNK, CHUNK)], sem))
    for c in copies:
        c.wait()

    wvs = [wv_v[pl.ds(c * LANES, LANES)] for c in range(EMB // LANES)]
    lanes = lax.iota(jnp.int32, LANES)
    w0v = w0b_v[0]
    biasv = w0b_v[1]

    UNROLL = 4

    def elem(i, carry):
        for u in range(UNROLL):
            ii = i * UNROLL + u
            p = None
            for c in range(EMB // LANES):
                ra = rows_a[ii, pl.ds(c * LANES, LANES)]
                rb = rows_b[ii, pl.ds(c * LANES, LANES)]
                t = ra * rb * wvs[c]
                p = t if p is None else p + t
            ps[ii, pl.ds(0, LANES)] = p
        return carry

    lax.fori_loop(0, BPW // UNROLL, elem, 0)

    def group(g, carry):
        ridx = lanes + g * LANES
        s = None
        for c in range(LANES):
            cc = jnp.full((LANES,), c, jnp.int32)
            col = plsc.load_gather(ps, [ridx, cc])
            s = col if s is None else s + col
        bav = ba[pl.ds(g * LANES, LANES)]
        bbv = bb[pl.ds(g * LANES, LANES)]
        acc = (bav + bbv) * w0v + biasv + s
        out_v[pl.ds(g * LANES, LANES)] = 1.0 / (1.0 + jnp.exp(-acc))
        return carry

    lax.fori_loop(0, NGROUP, group, 0)
    pltpu.sync_copy(out_v, out_ref.at[pl.ds(base, BPW)])


@functools.partial(jax.jit, static_argnames=())
def kernel(a, b, emb_table, emb_b_table, W, bias):
    a2 = a.astype(jnp.int32).reshape(NW * NCHUNK, CHUNK)
    b2 = b.astype(jnp.int32).reshape(NW * NCHUNK, CHUNK)
    wv = W[1:, 0]
    w0b = jnp.stack([jnp.full((LANES,), W[0, 0], jnp.float32),
                     jnp.full((LANES,), bias[0], jnp.float32)])

    mesh = plsc.VectorSubcoreMesh(core_axis_name="c", subcore_axis_name="s")
    run = pl.kernel(
        _amf_body,
        out_type=jax.ShapeDtypeStruct((BATCH,), jnp.float32),
        mesh=mesh,
        scratch_types=[
            pltpu.VMEM((NCHUNK, CHUNK), jnp.int32),
            pltpu.VMEM((NCHUNK, CHUNK), jnp.int32),
            pltpu.VMEM((BPW, EMB), jnp.float32),
            pltpu.VMEM((BPW, EMB), jnp.float32),
            pltpu.VMEM((BPW,), jnp.float32),
            pltpu.VMEM((BPW,), jnp.float32),
            pltpu.VMEM((EMB,), jnp.float32),
            pltpu.VMEM((2, LANES), jnp.float32),
            pltpu.VMEM((BPW, PSTRIDE), jnp.float32),
            pltpu.VMEM((BPW,), jnp.float32),
            pltpu.SemaphoreType.DMA,
        ],
        compiler_params=pltpu.CompilerParams(
            needs_layout_passes=False, use_tc_tiling_on_sc=False),
    )
    out = run(a2, b2, emb_table, emb_b_table.reshape(-1), wv, w0b)
    return out.reshape(BATCH, 1)
